# Initial kernel scaffold; baseline (speedup 1.0000x reference)
import jax
import jax.numpy as jnp
from jax.experimental import pallas as pl


def kernel(x, f, b, max_neighbors, W_self, W_msg, W_pos, b_fe, W_tf, b_tf, W_wf, b_wf, ln_g, ln_b, W_post, b_post):
    raise NotImplementedError("write your pallas kernel here")


# speedup vs baseline: 13.2051x; 13.2051x over previous
import jax
import jax.numpy as jnp
from jax.experimental import pallas as pl
from jax.experimental.pallas import tpu as pltpu

N = 10000
NP = 10240
D = 128
K = 16
M = 500
MP = 512
RB = 256
SUB = 8
LANE = NP // SUB
BIG = float("inf")


def _silu(v):
    return v * jax.nn.sigmoid(v)


def _fe_body(xb_ref, xall_ref, xt_ref, f_ref, wmsg_ref, wpos_ref, wself_ref,
             bfe_ref, h_ref, d2o_ref, wk_ref):
    i = pl.program_id(0)
    xb = xb_ref[...]
    xt = xt_ref[...]
    qn = jnp.sum(xb * xb, axis=1, keepdims=True)
    pn = jnp.sum(xt * xt, axis=0, keepdims=True)
    qp = jnp.dot(xb, xt, preferred_element_type=jnp.float32)
    d2 = (qn + pn) - 2.0 * qp
    col = jax.lax.broadcasted_iota(jnp.int32, (RB, NP), 1)
    d2 = jnp.where(col < N, d2, BIG)
    d2o_ref[...] = d2
    wk_ref[...] = d2

    def step(_, carry):
        w = wk_ref[...]
        m = jnp.min(w, axis=1, keepdims=True)
        wk_ref[...] = jnp.where(w <= m, BIG, w)
        return carry

    jax.lax.fori_loop(0, K - 1, step, 0)
    v16 = jnp.min(wk_ref[...], axis=1, keepdims=True)
    maskf = (d2o_ref[...] <= v16).astype(jnp.float32)

    sum_f = jnp.dot(maskf, f_ref[...], preferred_element_type=jnp.float32)
    sum_x = jnp.dot(maskf, xall_ref[...], preferred_element_type=jnp.float32)
    rel = sum_x - float(K) * xb
    msg = (jnp.dot(sum_f, wmsg_ref[...], preferred_element_type=jnp.float32)
           + jnp.dot(rel, wpos_ref[...], preferred_element_type=jnp.float32)
           ) * (1.0 / K)
    fb = f_ref[pl.ds(i * RB, RB), :]
    pre = jnp.dot(fb, wself_ref[...], preferred_element_type=jnp.float32)
    h_ref[...] = _silu(pre + msg + bfe_ref[...])


def _run_fe(xp, xt, fp, W_msg, W_pos, W_self, b_fe):
    grid = NP // RB
    return pl.pallas_call(
        _fe_body,
        grid=(grid,),
        in_specs=[
            pl.BlockSpec((RB, 3), lambda i: (i, 0)),
            pl.BlockSpec((NP, 3), lambda i: (0, 0)),
            pl.BlockSpec((3, NP), lambda i: (0, 0)),
            pl.BlockSpec((NP, D), lambda i: (0, 0)),
            pl.BlockSpec((D, D), lambda i: (0, 0)),
            pl.BlockSpec((3, D), lambda i: (0, 0)),
            pl.BlockSpec((D, D), lambda i: (0, 0)),
            pl.BlockSpec((1, D), lambda i: (0, 0)),
        ],
        out_specs=pl.BlockSpec((RB, D), lambda i: (i, 0)),
        out_shape=jax.ShapeDtypeStruct((NP, D), jnp.float32),
        scratch_shapes=[
            pltpu.VMEM((RB, NP), jnp.float32),
            pltpu.VMEM((RB, NP), jnp.float32),
        ],
    )(xp, xp, xt, fp, W_msg, W_pos, W_self, b_fe.reshape(1, D))


def _fps_body(xr_ref, qx_ref, qidx_ref):
    x0 = xr_ref[0]
    x1 = xr_ref[1]
    x2 = xr_ref[2]
    row = jax.lax.broadcasted_iota(jnp.int32, (SUB, LANE), 0)
    colf = jax.lax.broadcasted_iota(jnp.int32, (SUB, LANE), 1)
    flat = row * LANE + colf
    dmin0 = jnp.where(flat < N, BIG, -1.0)
    qidx_ref[0, 0] = 0

    def coords(idx):
        eq = flat == idx
        lx = jnp.max(jnp.where(eq, x0, -BIG))
        ly = jnp.max(jnp.where(eq, x1, -BIG))
        lz = jnp.max(jnp.where(eq, x2, -BIG))
        return lx, ly, lz

    def body(i, carry):
        dmin, prev = carry
        lx, ly, lz = coords(prev)
        qx_ref[i - 1, 0] = lx
        qx_ref[i - 1, 1] = ly
        qx_ref[i - 1, 2] = lz
        dx = x0 - lx
        dy = x1 - ly
        dz = x2 - lz
        d = (dx * dx + dy * dy) + dz * dz
        dmin = jnp.minimum(dmin, d)
        m = jnp.max(dmin)
        nxt = jnp.min(jnp.where(dmin == m, flat, 2 ** 30))
        qidx_ref[0, i] = nxt
        return dmin, nxt

    _, last = jax.lax.fori_loop(1, M, body, (dmin0, jnp.int32(0)))
    lx, ly, lz = coords(last)
    qx_ref[M - 1, 0] = lx
    qx_ref[M - 1, 1] = ly
    qx_ref[M - 1, 2] = lz
    for r in range(M, MP):
        qx_ref[r, 0] = 0.0
        qx_ref[r, 1] = 0.0
        qx_ref[r, 2] = 0.0
        qidx_ref[0, r] = 0


def _run_fps(xr):
    return pl.pallas_call(
        _fps_body,
        out_specs=(
            pl.BlockSpec(memory_space=pltpu.SMEM),
            pl.BlockSpec(memory_space=pltpu.SMEM),
        ),
        out_shape=(
            jax.ShapeDtypeStruct((MP, 3), jnp.float32),
            jax.ShapeDtypeStruct((1, MP), jnp.int32),
        ),
    )(xr)


def _tf_body(qxb_ref, xt_ref, h_ref, wtf_ref, btf_ref, wwf_ref, bwf_ref,
             lng_ref, lnb_ref, wpt_ref, bpost_ref, outf_ref, w_ref,
             d2o_ref, wk_ref):
    qxb = qxb_ref[...]
    xt = xt_ref[...]
    qn = jnp.sum(qxb * qxb, axis=1, keepdims=True)
    pn = jnp.sum(xt * xt, axis=0, keepdims=True)
    qp = jnp.dot(qxb, xt, preferred_element_type=jnp.float32)
    d2k = (qn + pn) - 2.0 * qp
    col = jax.lax.broadcasted_iota(jnp.int32, (RB, NP), 1)
    d2k = jnp.where(col < N, d2k, BIG)
    d2o_ref[...] = d2k
    wk_ref[...] = d2k

    def step(_, carry):
        w = wk_ref[...]
        m = jnp.min(w, axis=1, keepdims=True)
        wk_ref[...] = jnp.where(w <= m, BIG, w)
        return carry

    jax.lax.fori_loop(0, K - 1, step, 0)
    v16 = jnp.min(wk_ref[...], axis=1, keepdims=True)
    mask = d2o_ref[...] <= v16

    dxx = qxb[:, 0:1] - xt[0:1, :]
    dyy = qxb[:, 1:2] - xt[1:2, :]
    dzz = qxb[:, 2:3] - xt[2:3, :]
    d2d = (dxx * dxx + dyy * dyy) + dzz * dzz
    a = jnp.where(mask, -d2d, -BIG)
    amax = jnp.max(a, axis=1, keepdims=True)
    e = jnp.where(mask, jnp.exp(a - amax), 0.0)
    s = jnp.sum(e, axis=1, keepdims=True)
    attn = e / s

    agg = jnp.dot(attn, h_ref[...], preferred_element_type=jnp.float32)
    outf_ref[...] = (jnp.dot(agg, wtf_ref[...], preferred_element_type=jnp.float32)
                     + btf_ref[...])
    wpre = (jnp.dot(agg, wwf_ref[...], preferred_element_type=jnp.float32)
            + bwf_ref[...])
    mu = jnp.mean(wpre, axis=1, keepdims=True)
    ctr = wpre - mu
    var = jnp.mean(ctr * ctr, axis=1, keepdims=True)
    wn = ctr / jnp.sqrt(var + 1e-5) * lng_ref[...] + lnb_ref[...]
    sw = _silu(wn)
    logit = jnp.sum(sw * wpt_ref[...], axis=1, keepdims=True) + bpost_ref[0, 0]
    w_ref[...] = jax.nn.sigmoid(logit)


def _run_tf(qx_pad, xt, h, W_tf, b_tf, W_wf, b_wf, ln_g, ln_b, W_post, b_post):
    grid = MP // RB
    return pl.pallas_call(
        _tf_body,
        grid=(grid,),
        in_specs=[
            pl.BlockSpec((RB, 3), lambda i: (i, 0)),
            pl.BlockSpec((3, NP), lambda i: (0, 0)),
            pl.BlockSpec((NP, D), lambda i: (0, 0)),
            pl.BlockSpec((D, D), lambda i: (0, 0)),
            pl.BlockSpec((1, D), lambda i: (0, 0)),
            pl.BlockSpec((D, D), lambda i: (0, 0)),
            pl.BlockSpec((1, D), lambda i: (0, 0)),
            pl.BlockSpec((1, D), lambda i: (0, 0)),
            pl.BlockSpec((1, D), lambda i: (0, 0)),
            pl.BlockSpec((1, D), lambda i: (0, 0)),
            pl.BlockSpec((1, 1), lambda i: (0, 0)),
        ],
        out_specs=(
            pl.BlockSpec((RB, D), lambda i: (i, 0)),
            pl.BlockSpec((RB, 1), lambda i: (i, 0)),
        ),
        out_shape=(
            jax.ShapeDtypeStruct((MP, D), jnp.float32),
            jax.ShapeDtypeStruct((MP, 1), jnp.float32),
        ),
        scratch_shapes=[
            pltpu.VMEM((RB, NP), jnp.float32),
            pltpu.VMEM((RB, NP), jnp.float32),
        ],
    )(qx_pad, xt, h, W_tf, b_tf.reshape(1, D), W_wf, b_wf.reshape(1, D),
      ln_g.reshape(1, D), ln_b.reshape(1, D), W_post.reshape(1, D),
      b_post.reshape(1, 1))


def kernel(x, f, b, max_neighbors, W_self, W_msg, W_pos, b_fe, W_tf, b_tf,
           W_wf, b_wf, ln_g, ln_b, W_post, b_post):
    xp = jnp.pad(x, ((0, NP - N), (0, 0)))
    fp = jnp.pad(f, ((0, NP - N), (0, 0)))
    xt = xp.T
    xr = xt.reshape(3, SUB, LANE)

    h = _run_fe(xp, xt, fp, W_msg, W_pos, W_self, b_fe)
    qx_pad, qidx_pad = _run_fps(xr)
    out_f_pad, w_pad = _run_tf(qx_pad, xt, h, W_tf, b_tf, W_wf, b_wf,
                               ln_g, ln_b, W_post, b_post)

    qidx = qidx_pad[0, :M]
    qx = qx_pad[:M]
    qb = jnp.take(b, qidx)
    out_f = out_f_pad[:M]
    w = w_pad[:M, 0]
    return qx, out_f, qb, w
